# baseline (speedup 1.0000x reference)
import functools

import jax
import jax.numpy as jnp
from jax import lax
from jax.experimental import pallas as pl
from jax.experimental.pallas import tpu as pltpu
from jax.experimental.pallas import tpu_sc as plsc

_EMBED = 64
_NC = 2
_NS = 16
_NW = _NC * _NS
_L = 16
_W = 512


@functools.lru_cache(maxsize=None)
def _make_relayout(V: int):
    n_blocks = V // _W
    n_iters = -(-n_blocks // _NW)
    tail = V - n_blocks * _W
    pitch = _W + 1
    mesh = plsc.VectorSubcoreMesh(core_axis_name="c", subcore_axis_name="s")

    @functools.partial(
        pl.kernel,
        mesh=mesh,
        out_type=jax.ShapeDtypeStruct((V // 2, 2 * _EMBED), jnp.float32),
        scratch_types=[
            pltpu.VMEM((_EMBED, pitch), jnp.float32),
            pltpu.VMEM((_W // 2, 2 * _EMBED), jnp.float32),
            pltpu.SemaphoreType.DMA,
        ],
        compiler_params=pltpu.CompilerParams(use_tc_tiling_on_sc=True,
                                             needs_layout_passes=False,
                                             disable_bounds_checks=True),
    )
    def relayout(tT_hbm, tail_hbm, out_hbm, buf_in, buf_out, sem):
        wid = lax.axis_index("s") * _NC + lax.axis_index("c")
        rows = [jax.lax.iota(jnp.int32, _L) + _L * m for m in range(4)]

        def body(t, carry):
            g = wid + t * _NW

            @pl.when(g < n_blocks)
            def _():
                c0 = pl.multiple_of(g * _W, 128)
                o0 = pl.multiple_of(g * (_W // 2), 8)
                pltpu.sync_copy(tT_hbm.at[:, pl.ds(c0, _W)],
                                buf_in.at[:, pl.ds(0, _W)])

                for q in range(2 * _EMBED // _L):
                    init = jnp.full((_L,), q // 4, jnp.int32)

                    @plsc.parallel_loop(0, _W // 2, unroll=16, carry=init)
                    def trq(r, col):
                        vals = plsc.load_gather(buf_in, [rows[q % 4], col])
                        buf_out[r, pl.ds(q * _L, _L)] = vals
                        return col + 2
                pltpu.sync_copy(buf_out, out_hbm.at[pl.ds(o0, _W // 2)])

            return carry

        lax.fori_loop(0, n_iters, body, 0)

        @pl.when(wid == 0)
        def _():
            pltpu.sync_copy(tail_hbm, buf_out.at[pl.ds(0, tail // 2)])
            pltpu.sync_copy(buf_out.at[pl.ds(0, tail // 2)],
                            out_hbm.at[pl.ds(n_blocks * (_W // 2), tail // 2)])

    return relayout


@functools.lru_cache(maxsize=None)
def _make_gather(B: int, V: int):
    b_per_w = B // _NW
    C = 640
    n_chunks = b_per_w // C
    mesh = plsc.VectorSubcoreMesh(core_axis_name="c", subcore_axis_name="s")

    @functools.partial(
        pl.kernel,
        mesh=mesh,
        out_type=jax.ShapeDtypeStruct((B, 2 * _EMBED), jnp.float32),
        scratch_types=[
            pltpu.VMEM((C,), jnp.int32),
            pltpu.VMEM((C, _EMBED), jnp.float32),
            pltpu.SemaphoreType.DMA,
        ],
        compiler_params=pltpu.CompilerParams(use_tc_tiling_on_sc=False,
                                             disable_bounds_checks=True),
    )
    def gather(idx_hbm, table_hbm, out_hbm, idx_v, rows_v, sem):
        wid = lax.axis_index("s") * _NC + lax.axis_index("c")
        base = wid * b_per_w

        def body(j, carry):
            off = base + j * C
            pltpu.sync_copy(idx_hbm.at[pl.ds(off, C)], idx_v)
            pltpu.async_copy(table_hbm.at[idx_v], rows_v, sem).wait()
            pltpu.sync_copy(rows_v, out_hbm.at[pl.ds(off, C), pl.ds(0, _EMBED)])
            return carry

        lax.fori_loop(0, n_chunks, body, 0)

    return gather


def kernel(x, table):
    B = x.shape[0] * x.shape[1]
    V = table.shape[0]
    n_main = (V // _W) * _W
    t_tail = table[n_main:].reshape(-1, 2 * _EMBED)
    t_lin = _make_relayout(V)(table.T, t_tail)
    out128 = _make_gather(B, V)(x.reshape(B), t_lin.reshape(V, _EMBED))
    return out128[:, :_EMBED].reshape(x.shape[0], x.shape[1], _EMBED)


# speedup vs baseline: 1.5444x; 1.5444x over previous
import functools

import jax
import jax.numpy as jnp
from jax import lax
from jax.experimental import pallas as pl
from jax.experimental.pallas import tpu as pltpu
from jax.experimental.pallas import tpu_sc as plsc

_EMBED = 64
_NC = 2
_NS = 16
_NW = _NC * _NS


@functools.lru_cache(maxsize=None)
def _make_gather(B: int):
    b_per_w = B // _NW
    C = 800
    n = b_per_w // C
    mesh = plsc.VectorSubcoreMesh(core_axis_name="c", subcore_axis_name="s")

    @functools.partial(
        pl.kernel,
        mesh=mesh,
        out_type=jax.ShapeDtypeStruct((B, 2 * _EMBED), jnp.float32),
        scratch_types=[
            pltpu.VMEM((C,), jnp.int32),
            pltpu.VMEM((C,), jnp.int32),
            pltpu.VMEM((C, _EMBED), jnp.float32),
            pltpu.VMEM((C, _EMBED), jnp.float32),
            pltpu.SemaphoreType.DMA,
            pltpu.SemaphoreType.DMA,
            pltpu.SemaphoreType.DMA,
            pltpu.SemaphoreType.DMA,
        ],
        compiler_params=pltpu.CompilerParams(use_tc_tiling_on_sc=False),
    )
    def gather(idx_hbm, table_hbm, out_hbm, i0, i1, r0, r1, sg0, sg1, st0, st1):
        wid = lax.axis_index("s") * _NC + lax.axis_index("c")
        base = wid * b_per_w
        idx_bufs = (i0, i1)
        row_bufs = (r0, r1)
        g_sems = (sg0, sg1)
        s_sems = (st0, st1)

        def idx_load(k, slot):
            pltpu.sync_copy(idx_hbm.at[pl.ds(base + k * C, C)], idx_bufs[slot])

        def gather_start(slot):
            pltpu.async_copy(table_hbm.at[idx_bufs[slot]], row_bufs[slot],
                             g_sems[slot])

        def gather_wait(slot):
            pltpu.make_async_copy(table_hbm.at[idx_bufs[slot]], row_bufs[slot],
                                  g_sems[slot]).wait()

        def out_slice(k):
            return out_hbm.at[pl.ds(base + k * C, C), pl.ds(0, _EMBED)]

        def store_start(k, slot):
            pltpu.async_copy(row_bufs[slot], out_slice(k), s_sems[slot])

        def store_wait(k, slot):
            pltpu.make_async_copy(row_bufs[slot], out_slice(k),
                                  s_sems[slot]).wait()

        idx_load(0, 0)
        gather_start(0)
        idx_load(1, 1)
        gather_wait(0)
        store_start(0, 0)
        idx_load(2, 0)
        gather_start(1)

        def body(t, carry):
            j1 = 2 * t + 1
            j2 = j1 + 1
            gather_wait(1)
            store_start(j1, 1)
            idx_load(jnp.minimum(j1 + 2, n - 1), 1)
            store_wait(j1 - 1, 0)
            gather_start(0)
            gather_wait(0)
            store_start(j2, 0)
            idx_load(jnp.minimum(j2 + 2, n - 1), 0)
            store_wait(j2 - 1, 1)
            gather_start(1)
            return carry

        lax.fori_loop(0, (n - 2) // 2, body, 0)

        gather_wait(1)
        store_start(n - 1, 1)
        store_wait(n - 2, 0)
        store_wait(n - 1, 1)

    return gather


def kernel(x, table):
    B = x.shape[0] * x.shape[1]
    out128 = _make_gather(B)(x.reshape(B), table)
    return out128[:, :_EMBED].reshape(x.shape[0], x.shape[1], _EMBED)
